# Initial kernel scaffold; baseline (speedup 1.0000x reference)
import functools

import jax
import jax.numpy as jnp
from jax import lax
from jax.experimental import pallas as pl
from jax.experimental.pallas import tpu as pltpu
from jax.experimental.pallas import tpu_sc as plsc

N_NODES = 10000
NUM_CORES = 2
NUM_SUBCORES = 16
NUM_WORKERS = NUM_CORES * NUM_SUBCORES
CHUNK = 128
ROWS_PER_SUB = N_NODES // NUM_SUBCORES


@functools.lru_cache(maxsize=None)
def _make_spmm(feat: int, n_chunks_w: int):
    mesh = plsc.VectorSubcoreMesh(core_axis_name="c", subcore_axis_name="s")

    @functools.partial(
        pl.kernel,
        out_type=jax.ShapeDtypeStruct((NUM_CORES, N_NODES, feat), jnp.float32),
        mesh=mesh,
        scratch_types=[
            pltpu.VMEM((n_chunks_w, CHUNK), jnp.int32),
            pltpu.VMEM((n_chunks_w, CHUNK), jnp.int32),
            pltpu.VMEM((n_chunks_w, CHUNK), jnp.float32),
            pltpu.VMEM((CHUNK, feat), jnp.float32),
            pltpu.VMEM_SHARED((N_NODES, feat), jnp.float32),
            pltpu.SemaphoreType.DMA,
        ],
    )
    def spmm(sup_hbm, src_hbm, dst_hbm, w_hbm, zeros_hbm, out_hbm,
             src_v, dst_v, w_v, rows_v, acc, sem):
        c = lax.axis_index("c")
        s = lax.axis_index("s")
        wid = c * NUM_SUBCORES + s

        pltpu.sync_copy(zeros_hbm, acc.at[pl.ds(s * ROWS_PER_SUB, ROWS_PER_SUB)])

        base = wid * n_chunks_w
        pltpu.sync_copy(src_hbm.at[pl.ds(base, n_chunks_w)], src_v)
        pltpu.sync_copy(dst_hbm.at[pl.ds(base, n_chunks_w)], dst_v)
        pltpu.sync_copy(w_hbm.at[pl.ds(base, n_chunks_w)], w_v)

        plsc.subcore_barrier()

        def chunk_body(ch, carry):
            pltpu.async_copy(sup_hbm.at[src_v.at[ch]], rows_v, sem).wait()

            def edge_body(e, carry2):
                we = w_v[ch, e]
                for f in range(feat // 16):
                    sl = pl.ds(f * 16, 16)
                    rows_v[e, sl] = rows_v[e, sl] * we
                return carry2

            lax.fori_loop(0, CHUNK, edge_body, 0)

            pltpu.sync_copy(rows_v, acc.at[dst_v.at[ch]], add=True)
            return carry

        lax.fori_loop(0, n_chunks_w, chunk_body, 0)

        plsc.subcore_barrier()

        pltpu.sync_copy(acc.at[pl.ds(s * ROWS_PER_SUB, ROWS_PER_SUB)],
                        out_hbm.at[c, pl.ds(s * ROWS_PER_SUB, ROWS_PER_SUB)])

    return spmm


_BLK = 1000


def _tc_matmul(x, w):
    n, k = x.shape
    m = w.shape[1]

    def body(x_ref, w_ref, o_ref):
        o_ref[...] = jnp.dot(x_ref[...], w_ref[...],
                             preferred_element_type=jnp.float32)

    return pl.pallas_call(
        body,
        grid=(n // _BLK,),
        in_specs=[pl.BlockSpec((_BLK, k), lambda i: (i, 0)),
                  pl.BlockSpec((k, m), lambda i: (0, 0))],
        out_specs=pl.BlockSpec((_BLK, m), lambda i: (i, 0)),
        out_shape=jax.ShapeDtypeStruct((n, m), jnp.float32),
    )(x, w)


def _tc_relu_matmul(p0, p1, b, w):
    n, k = p0.shape
    m = w.shape[1]

    def body(p0_ref, p1_ref, b_ref, w_ref, o_ref):
        h = jnp.maximum(p0_ref[...] + p1_ref[...] + b_ref[...], 0.0)
        o_ref[...] = jnp.dot(h, w_ref[...], preferred_element_type=jnp.float32)

    return pl.pallas_call(
        body,
        grid=(n // _BLK,),
        in_specs=[pl.BlockSpec((_BLK, k), lambda i: (i, 0)),
                  pl.BlockSpec((_BLK, k), lambda i: (i, 0)),
                  pl.BlockSpec((1, k), lambda i: (0, 0)),
                  pl.BlockSpec((k, m), lambda i: (0, 0))],
        out_specs=pl.BlockSpec((_BLK, m), lambda i: (i, 0)),
        out_shape=jax.ShapeDtypeStruct((n, m), jnp.float32),
    )(p0, p1, b, w)


def _tc_bias_softmax(p0, p1, b):
    n, m = p0.shape

    def body(p0_ref, p1_ref, b_ref, o_ref):
        z = p0_ref[...] + p1_ref[...] + b_ref[...]
        z = z - jnp.max(z, axis=1, keepdims=True)
        e = jnp.exp(z)
        o_ref[...] = e / jnp.sum(e, axis=1, keepdims=True)

    return pl.pallas_call(
        body,
        grid=(n // _BLK,),
        in_specs=[pl.BlockSpec((_BLK, m), lambda i: (i, 0)),
                  pl.BlockSpec((_BLK, m), lambda i: (i, 0)),
                  pl.BlockSpec((1, m), lambda i: (0, 0))],
        out_specs=pl.BlockSpec((_BLK, m), lambda i: (i, 0)),
        out_shape=jax.ShapeDtypeStruct((n, m), jnp.float32),
    )(p0, p1, b)


def kernel(x, edge_index, edge_weight, W1, b1, W2, b2):
    src = edge_index[0].astype(jnp.int32)
    dst = edge_index[1].astype(jnp.int32)
    ew = edge_weight.astype(jnp.float32)

    e = src.shape[0]
    tile = NUM_WORKERS * CHUNK
    e_pad = ((e + tile - 1) // tile) * tile
    pad = e_pad - e
    src_p = jnp.pad(src, (0, pad)).reshape(e_pad // CHUNK, CHUNK)
    dst_p = jnp.pad(dst, (0, pad)).reshape(e_pad // CHUNK, CHUNK)
    ew_p = jnp.pad(ew, (0, pad)).reshape(e_pad // CHUNK, CHUNK)
    n_chunks_w = e_pad // (NUM_WORKERS * CHUNK)

    nhid = W1.shape[1]
    ncls = W2.shape[1]
    zeros_h = jnp.zeros((ROWS_PER_SUB, nhid), jnp.float32)
    zeros_c = jnp.zeros((ROWS_PER_SUB, ncls), jnp.float32)

    support = _tc_matmul(x, W1)
    parts1 = _make_spmm(nhid, n_chunks_w)(support, src_p, dst_p, ew_p, zeros_h)
    support2 = _tc_relu_matmul(parts1[0], parts1[1], b1.reshape(1, -1), W2)
    parts2 = _make_spmm(ncls, n_chunks_w)(support2, src_p, dst_p, ew_p, zeros_c)
    return _tc_bias_softmax(parts2[0], parts2[1], b2.reshape(1, -1))


# speedup vs baseline: 2.8379x; 2.8379x over previous
import functools

import jax
import jax.numpy as jnp
from jax import lax
from jax.experimental import pallas as pl
from jax.experimental.pallas import tpu as pltpu
from jax.experimental.pallas import tpu_sc as plsc

N_NODES = 10000
NUM_CORES = 2
NUM_SUBCORES = 16
NUM_WORKERS = NUM_CORES * NUM_SUBCORES
CHUNK = 128
N_PAD = 10240
ROWS_PER_SUB = N_PAD // NUM_SUBCORES


@functools.lru_cache(maxsize=None)
def _make_spmm(feat: int, n_chunks_w: int):
    mesh = plsc.VectorSubcoreMesh(core_axis_name="c", subcore_axis_name="s")

    @functools.partial(
        pl.kernel,
        out_type=jax.ShapeDtypeStruct((NUM_CORES, N_PAD, feat), jnp.float32),
        mesh=mesh,
        scratch_types=[
            pltpu.VMEM((n_chunks_w, CHUNK), jnp.int32),
            pltpu.VMEM((n_chunks_w, CHUNK), jnp.int32),
            pltpu.VMEM((n_chunks_w, CHUNK), jnp.float32),
            pltpu.VMEM((CHUNK, feat), jnp.float32),
            pltpu.VMEM_SHARED((N_PAD, feat), jnp.float32),
            pltpu.SemaphoreType.DMA,
        ],
    )
    def spmm(sup_hbm, src_hbm, dst_hbm, w_hbm, zeros_hbm, out_hbm,
             src_v, dst_v, w_v, rows_v, acc, sem):
        c = lax.axis_index("c")
        s = lax.axis_index("s")
        wid = c * NUM_SUBCORES + s

        pltpu.sync_copy(zeros_hbm, acc.at[pl.ds(s * ROWS_PER_SUB, ROWS_PER_SUB)])

        base = wid * n_chunks_w
        pltpu.sync_copy(src_hbm.at[pl.ds(base, n_chunks_w)], src_v)
        pltpu.sync_copy(dst_hbm.at[pl.ds(base, n_chunks_w)], dst_v)
        pltpu.sync_copy(w_hbm.at[pl.ds(base, n_chunks_w)], w_v)

        plsc.subcore_barrier()

        def chunk_body(ch, carry):
            pltpu.async_copy(sup_hbm.at[src_v.at[ch]], rows_v, sem).wait()

            def group_body(g, carry2):
                w16 = w_v[ch, pl.ds(g * 16, 16)]
                for j in range(16):
                    we = w16[j]
                    e = g * 16 + j
                    for f in range(feat // 16):
                        sl = pl.ds(f * 16, 16)
                        rows_v[e, sl] = rows_v[e, sl] * we
                return carry2

            lax.fori_loop(0, CHUNK // 16, group_body, 0)

            pltpu.sync_copy(rows_v, acc.at[dst_v.at[ch]], add=True)
            return carry

        lax.fori_loop(0, n_chunks_w, chunk_body, 0)

        plsc.subcore_barrier()

        pltpu.sync_copy(acc.at[pl.ds(s * ROWS_PER_SUB, ROWS_PER_SUB)],
                        out_hbm.at[c, pl.ds(s * ROWS_PER_SUB, ROWS_PER_SUB)])

    return spmm


def _blk(n):
    return 1024 if n % 1024 == 0 else 1000


def _tc_matmul(x, w):
    n, k = x.shape
    m = w.shape[1]
    _BLK = _blk(n)

    def body(x_ref, w_ref, o_ref):
        o_ref[...] = jnp.dot(x_ref[...], w_ref[...],
                             preferred_element_type=jnp.float32)

    return pl.pallas_call(
        body,
        grid=(n // _BLK,),
        in_specs=[pl.BlockSpec((_BLK, k), lambda i: (i, 0)),
                  pl.BlockSpec((k, m), lambda i: (0, 0))],
        out_specs=pl.BlockSpec((_BLK, m), lambda i: (i, 0)),
        out_shape=jax.ShapeDtypeStruct((n, m), jnp.float32),
    )(x, w)


def _tc_add_relu_matmul(p0, p1, b, w):
    n, k = p0.shape
    m = w.shape[1]
    _BLK = _blk(n)

    def body(p0_ref, p1_ref, b_ref, w_ref, o_ref):
        h = jnp.maximum(p0_ref[...] + p1_ref[...] + b_ref[...], 0.0)
        o_ref[...] = jnp.dot(h, w_ref[...], preferred_element_type=jnp.float32)

    return pl.pallas_call(
        body,
        grid=(n // _BLK,),
        in_specs=[pl.BlockSpec((_BLK, k), lambda i: (i, 0)),
                  pl.BlockSpec((_BLK, k), lambda i: (i, 0)),
                  pl.BlockSpec((1, k), lambda i: (0, 0)),
                  pl.BlockSpec((k, m), lambda i: (0, 0))],
        out_specs=pl.BlockSpec((_BLK, m), lambda i: (i, 0)),
        out_shape=jax.ShapeDtypeStruct((n, m), jnp.float32),
    )(p0, p1, b, w)


def _tc_bias_softmax(p0, p1, b, ncls):
    n, k = p0.shape
    _BLK = _blk(n)

    def body(p0_ref, p1_ref, b_ref, o_ref):
        z = p0_ref[:, :ncls] + p1_ref[:, :ncls] + b_ref[...]
        z = z - jnp.max(z, axis=1, keepdims=True)
        e = jnp.exp(z)
        o_ref[...] = e / jnp.sum(e, axis=1, keepdims=True)

    return pl.pallas_call(
        body,
        grid=(n // _BLK,),
        in_specs=[pl.BlockSpec((_BLK, k), lambda i: (i, 0)),
                  pl.BlockSpec((_BLK, k), lambda i: (i, 0)),
                  pl.BlockSpec((1, ncls), lambda i: (0, 0))],
        out_specs=pl.BlockSpec((_BLK, ncls), lambda i: (i, 0)),
        out_shape=jax.ShapeDtypeStruct((n, ncls), jnp.float32),
    )(p0, p1, b)


def kernel(x, edge_index, edge_weight, W1, b1, W2, b2):
    src = edge_index[0].astype(jnp.int32)
    dst = edge_index[1].astype(jnp.int32)
    ew = edge_weight.astype(jnp.float32)

    e = src.shape[0]
    tile = NUM_WORKERS * CHUNK * 8
    e_pad = ((e + tile - 1) // tile) * tile
    pad = e_pad - e
    src_p = jnp.pad(src, (0, pad)).reshape(e_pad // CHUNK, CHUNK)
    dst_p = jnp.pad(dst, (0, pad)).reshape(e_pad // CHUNK, CHUNK)
    ew_p = jnp.pad(ew, (0, pad)).reshape(e_pad // CHUNK, CHUNK)
    n_chunks_w = e_pad // (NUM_WORKERS * CHUNK)

    nhid = W1.shape[1]
    ncls = W2.shape[1]
    zeros_h = jnp.zeros((ROWS_PER_SUB, nhid), jnp.float32)
    spmm = _make_spmm(nhid, n_chunks_w)
    w2_pad = jnp.pad(W2, ((0, 0), (0, nhid - ncls)))

    support = _tc_matmul(x, W1)
    parts1 = spmm(support, src_p, dst_p, ew_p, zeros_h)
    support2 = _tc_add_relu_matmul(parts1[0], parts1[1], b1.reshape(1, -1),
                                   w2_pad)
    parts2 = spmm(support2, src_p, dst_p, ew_p, zeros_h)
    out = _tc_bias_softmax(parts2[0], parts2[1], b2.reshape(1, -1), ncls)
    return out[:N_NODES]
